# baseline (speedup 1.0000x reference)
import functools

import jax
import jax.numpy as jnp
from jax import lax
from jax.experimental import pallas as pl
from jax.experimental.pallas import tpu as pltpu
from jax.experimental.pallas import tpu_sc as plsc

HIDDEN = 768
EPS = 1e-12
NC, NS = 2, 16
NW = NC * NS
LANES = 16
NSLICE = HIDDEN // LANES
CHUNK = 64


_GATHER_DNUMS = lax.GatherDimensionNumbers(
    offset_dims=(), collapsed_slice_dims=(0,), start_index_map=(0,))


def _shuffle(v, idx):
    return lax.gather(v, idx[:, None], _GATHER_DNUMS, slice_sizes=(1,),
                      mode=lax.GatherScatterMode.PROMISE_IN_BOUNDS)


def _lane_sum(v):
    iota = lax.iota(jnp.int32, LANES)
    for sh in (1, 2, 4, 8):
        v = v + _shuffle(v, iota ^ sh)
    return v


def _rsqrt16(x):
    s = x
    y = jnp.zeros_like(x) + 1.0
    for k in (16, 8, 4, 2, 1):
        hi = s >= (4.0 ** k)
        s = s * jnp.where(hi, 4.0 ** -k, 1.0)
        y = y * jnp.where(hi, 2.0 ** -k, 1.0)
        lo = s < (4.0 ** -k)
        s = s * jnp.where(lo, 4.0 ** k, 1.0)
        y = y * jnp.where(lo, 2.0 ** k, 1.0)
    r = 2.0 / (1.0 + s)
    for _ in range(4):
        r = r * (1.5 - 0.5 * s * r * r)
    return y * r


@functools.lru_cache(maxsize=None)
def _build(ntok, seq):
    tok_per_w = ntok // NW
    nchunk = tok_per_w // CHUNK
    mesh = plsc.VectorSubcoreMesh(core_axis_name="c", subcore_axis_name="s")

    @functools.partial(
        pl.kernel,
        mesh=mesh,
        out_type=jax.ShapeDtypeStruct((ntok, HIDDEN), jnp.float32),
        scratch_types=[
            pltpu.VMEM((nchunk, CHUNK), jnp.int32),
            pltpu.VMEM((CHUNK, HIDDEN), jnp.float32),
            pltpu.VMEM((CHUNK, HIDDEN), jnp.float32),
            pltpu.VMEM((HIDDEN,), jnp.float32),
            pltpu.VMEM((HIDDEN,), jnp.float32),
            pltpu.SemaphoreType.DMA,
        ],
    )
    def body(ids_hbm, tab_hbm, pos_hbm, gam_hbm, bet_hbm, out_hbm,
             idx_v, row_v, pos_v, gam_v, bet_v, sem):
        wid = lax.axis_index("s") * NC + lax.axis_index("c")
        tok0 = wid * tok_per_w
        p0 = lax.rem(tok0, seq)
        pltpu.sync_copy(ids_hbm.at[wid], idx_v)
        pltpu.sync_copy(gam_hbm, gam_v)
        pltpu.sync_copy(bet_hbm, bet_v)

        for g in range(nchunk):
            pltpu.async_copy(tab_hbm.at[idx_v.at[g]], row_v, sem).wait()
            pltpu.sync_copy(pos_hbm.at[pl.ds(p0 + g * CHUNK, CHUNK)], pos_v)

            def row_body(r, carry):
                s = jnp.zeros((LANES,), jnp.float32)
                q = jnp.zeros((LANES,), jnp.float32)
                for k in range(NSLICE):
                    x = (row_v[r, pl.ds(k * LANES, LANES)]
                         + pos_v[r, pl.ds(k * LANES, LANES)])
                    s = s + x
                    q = q + x * x
                mv = _lane_sum(s) * (1.0 / HIDDEN)
                var = _lane_sum(q) * (1.0 / HIDDEN) - mv * mv
                inv = _rsqrt16(var + EPS)
                for k in range(NSLICE):
                    sl = pl.ds(k * LANES, LANES)
                    x = row_v[r, sl] + pos_v[r, sl]
                    row_v[r, sl] = (x - mv) * inv * gam_v[sl] + bet_v[sl]
                return carry

            lax.fori_loop(0, CHUNK, row_body, 0)
            pltpu.sync_copy(row_v, out_hbm.at[pl.ds(tok0 + g * CHUNK, CHUNK)])

    return body


def kernel(input_ids, word_embeddings, position_embeddings, ln_gamma, ln_beta):
    b, s = input_ids.shape
    ntok = b * s
    ids = input_ids.astype(jnp.int32).reshape(NW, -1, CHUNK)
    out = _build(ntok, s)(ids, word_embeddings, position_embeddings,
                          ln_gamma, ln_beta)
    return out.reshape(b, s, HIDDEN)


# speedup vs baseline: 2.0862x; 2.0862x over previous
import functools

import jax
import jax.numpy as jnp
from jax import lax
from jax.experimental import pallas as pl
from jax.experimental.pallas import tpu as pltpu
from jax.experimental.pallas import tpu_sc as plsc

HIDDEN = 768
EPS = 1e-12
NC, NS = 2, 16
NW = NC * NS
CHUNK = 64
TCBLK = 256


@functools.lru_cache(maxsize=None)
def _build_gather(ntok):
    tok_per_w = ntok // NW
    nchunk = tok_per_w // CHUNK
    mesh = plsc.VectorSubcoreMesh(core_axis_name="c", subcore_axis_name="s")

    @functools.partial(
        pl.kernel,
        mesh=mesh,
        out_type=jax.ShapeDtypeStruct((ntok, HIDDEN), jnp.float32),
        scratch_types=[
            pltpu.VMEM((nchunk, CHUNK), jnp.int32),
            pltpu.VMEM((2, CHUNK, HIDDEN), jnp.float32),
            pltpu.SemaphoreType.DMA,
            pltpu.SemaphoreType.DMA,
        ],
    )
    def body(ids_hbm, tab_hbm, out_hbm, idx_v, buf_v, gsem, wsem):
        wid = lax.axis_index("s") * NC + lax.axis_index("c")
        tok0 = wid * tok_per_w
        pltpu.sync_copy(ids_hbm.at[wid], idx_v)

        cur_g = pltpu.async_copy(tab_hbm.at[idx_v.at[0]], buf_v.at[0], gsem)
        prev_w = None
        for g in range(nchunk):
            cur_g.wait()
            if prev_w is not None:
                prev_w.wait()
            prev_w = pltpu.async_copy(
                buf_v.at[g % 2], out_hbm.at[pl.ds(tok0 + g * CHUNK, CHUNK)],
                wsem)
            if g + 1 < nchunk:
                cur_g = pltpu.async_copy(
                    tab_hbm.at[idx_v.at[g + 1]], buf_v.at[(g + 1) % 2], gsem)
        prev_w.wait()

    return body


def _ln_body(rows_ref, pos_ref, gam_ref, bet_ref, out_ref):
    x = rows_ref[...] + pos_ref[...]
    m = jnp.mean(x, axis=-1, keepdims=True)
    xc = x - m
    var = jnp.mean(xc * xc, axis=-1, keepdims=True)
    out_ref[...] = xc * lax.rsqrt(var + EPS) * gam_ref[...] + bet_ref[...]


@functools.lru_cache(maxsize=None)
def _build_ln(ntok, seq):
    nposblk = seq // TCBLK
    return pl.pallas_call(
        _ln_body,
        grid=(ntok // TCBLK,),
        in_specs=[
            pl.BlockSpec((TCBLK, HIDDEN), lambda i: (i, 0)),
            pl.BlockSpec((TCBLK, HIDDEN), lambda i: (i % nposblk, 0)),
            pl.BlockSpec((1, HIDDEN), lambda i: (0, 0)),
            pl.BlockSpec((1, HIDDEN), lambda i: (0, 0)),
        ],
        out_specs=pl.BlockSpec((TCBLK, HIDDEN), lambda i: (i, 0)),
        out_shape=jax.ShapeDtypeStruct((ntok, HIDDEN), jnp.float32),
    )


def kernel(input_ids, word_embeddings, position_embeddings, ln_gamma, ln_beta):
    b, s = input_ids.shape
    ntok = b * s
    ids = input_ids.astype(jnp.int32).reshape(NW, -1, CHUNK)
    rows = _build_gather(ntok)(ids, word_embeddings)
    out = _build_ln(ntok, s)(rows, position_embeddings,
                             ln_gamma[None], ln_beta[None])
    return out.reshape(b, s, HIDDEN)
